# Initial kernel scaffold; baseline (speedup 1.0000x reference)
import jax
import jax.numpy as jnp
from jax.experimental import pallas as pl


def kernel(node_scalar, node_vector, edge_index, edge_rbf, edge_udiff, W1, b1, W2, b2, We, be, gamma, beta, Wg, bg):
    raise NotImplementedError("write your pallas kernel here")


# speedup vs baseline: 5.0177x; 5.0177x over previous
import functools
import math

import jax
import jax.numpy as jnp
from jax import lax
from jax.experimental import pallas as pl
from jax.experimental.pallas import tpu as pltpu
from jax.experimental.pallas import tpu_sc as plsc

F32 = jnp.float32
INV_SQRT_3 = 1.0 / math.sqrt(3.0)
SILU_SCALE = 1.0 / 0.6

NC = 2
NS = 16
NW = NC * NS

@functools.cache
def _sc_mesh():
    return plsc.VectorSubcoreMesh(
        core_axis_name="c", subcore_axis_name="s",
        num_cores=NC, num_subcores=NS)


def _ssilu(x):
    return jax.nn.silu(x) * SILU_SCALE


def _make_k1(e_pad, n_pad, blk, w):
    nblk = e_pad // NW // blk
    ew = e_pad // NW
    rows_t = n_pad // NS

    def body(idx_hbm, ud4_hbm, z4_hbm, out_hbm, idxv, udv, sem, acc4):
        core = lax.axis_index("c")
        sub = lax.axis_index("s")
        r0 = sub * rows_t
        pltpu.sync_copy(z4_hbm.at[pl.ds(r0, rows_t)], acc4.at[pl.ds(r0, rows_t)])
        plsc.subcore_barrier()
        wbase = core * (e_pad // NC) + sub * ew

        @pl.loop(0, nblk)
        def _(b):
            base = pl.multiple_of(wbase + b * blk, 8)
            pltpu.sync_copy(idx_hbm.at[pl.ds(base, blk)], idxv)
            pltpu.async_copy(ud4_hbm.at[pl.ds(base, blk)], udv, sem).wait()
            pltpu.sync_copy(udv, acc4.at[idxv], add=True)

        plsc.subcore_barrier()
        pltpu.sync_copy(acc4.at[pl.ds(r0, rows_t)],
                        out_hbm.at[core, pl.ds(r0, rows_t)])

    return pl.kernel(
        body,
        out_type=jax.ShapeDtypeStruct((NC, n_pad, w), F32),
        mesh=_sc_mesh(),
        compiler_params=pltpu.CompilerParams(use_tc_tiling_on_sc=False),
        scratch_types=[
            pltpu.VMEM((blk,), jnp.int32),
            pltpu.VMEM((blk, w), F32),
            pltpu.SemaphoreType.DMA,
            pltpu.VMEM_SHARED((n_pad, w), F32),
        ],
    )


def _make_k3(e_pad, blk, w):
    nblk = e_pad // NW // blk
    ew = e_pad // NW

    def body(idx_hbm, rv4_hbm, out_hbm, idxv, rvv, sem):
        core = lax.axis_index("c")
        sub = lax.axis_index("s")
        wbase = core * (e_pad // NC) + sub * ew

        @pl.loop(0, nblk)
        def _(b):
            base = pl.multiple_of(wbase + b * blk, 8)
            pltpu.sync_copy(idx_hbm.at[pl.ds(base, blk)], idxv)
            pltpu.async_copy(rv4_hbm.at[idxv], rvv, sem).wait()
            pltpu.sync_copy(rvv, out_hbm.at[pl.ds(base, blk)])

    return pl.kernel(
        body,
        out_type=jax.ShapeDtypeStruct((e_pad, w), F32),
        mesh=_sc_mesh(),
        compiler_params=pltpu.CompilerParams(use_tc_tiling_on_sc=False),
        scratch_types=[
            pltpu.VMEM((blk,), jnp.int32),
            pltpu.VMEM((blk, w), F32),
            pltpu.SemaphoreType.DMA,
        ],
    )


def _make_k5(e_pad, n, n_pad, h, blk):
    hh = h // 2
    ew = e_pad // NS
    nblk = ew // blk
    rows_t = n_pad // NS
    hv = hh // 16

    def body(idxj_hbm, idxi_hbm, sp1, sp2, sp3, nv0, nv1, nv2,
             c3, c1s, c20, c21, c22, z_hbm, outs_hbm, outv_hbm,
             idxjv, idxjv2, idxiv, av, bv, cv, dv, ev, outv,
             s0, s1, s2, s3, s4, acc):
        core = lax.axis_index("c")
        sub = lax.axis_index("s")
        r0 = sub * rows_t
        wbase = sub * ew
        last = n - (NS - 1) * rows_t
        joff = core * n
        eoff = core * e_pad

        chunks = (
            (sp3, None, c3, None, None),
            (sp1, sp2, c1s, c20, nv0),
            (sp1, sp2, c1s, c21, nv1),
            (sp1, sp2, c1s, c22, nv2),
        )
        for q, (tab1, tab2, co1, co2, nvt) in enumerate(chunks):
            pltpu.sync_copy(z_hbm.at[pl.ds(r0, rows_t)],
                            acc.at[pl.ds(r0, rows_t)])
            plsc.subcore_barrier()

            @pl.loop(0, nblk)
            def _(b):
                base = pl.multiple_of(wbase + b * blk, 8)
                cbase = pl.multiple_of(eoff + base, 8)
                pltpu.sync_copy(idxj_hbm.at[pl.ds(base, blk)], idxjv)
                pltpu.sync_copy(idxi_hbm.at[pl.ds(base, blk)], idxiv)

                @pl.loop(0, blk // 16)
                def _(k):
                    s = pl.ds(k * 16, 16)
                    idxjv2[s] = idxjv[s] + joff

                cp0 = pltpu.async_copy(tab1.at[idxjv2], av, s0)
                cp1 = pltpu.async_copy(co1.at[pl.ds(cbase, blk)], cv, s1)
                if tab2 is None:
                    cp0.wait()
                    cp1.wait()

                    @pl.loop(0, blk)
                    def _(r):
                        for ccol in range(hv):
                            s = pl.ds(ccol * 16, 16)
                            outv[r, s] = av[r, s] * cv[r, s]
                else:
                    cp2 = pltpu.async_copy(tab2.at[idxjv2], bv, s2)
                    cp3 = pltpu.async_copy(co2.at[pl.ds(cbase, blk)], dv, s3)
                    cp4 = pltpu.async_copy(nvt.at[idxjv2], ev, s4)
                    cp0.wait()
                    cp1.wait()
                    cp2.wait()
                    cp3.wait()
                    cp4.wait()

                    @pl.loop(0, blk)
                    def _(r):
                        for ccol in range(hv):
                            s = pl.ds(ccol * 16, 16)
                            outv[r, s] = (av[r, s] * cv[r, s] * ev[r, s]
                                          + bv[r, s] * dv[r, s])
                pltpu.sync_copy(outv, acc.at[idxiv], add=True)

            plsc.subcore_barrier()

            def _flush(nrows):
                if q == 0:
                    pltpu.sync_copy(
                        acc.at[pl.ds(r0, nrows)],
                        outs_hbm.at[core, pl.ds(r0, nrows)])
                else:
                    pltpu.sync_copy(
                        acc.at[pl.ds(r0, nrows)],
                        outv_hbm.at[q - 1, core, pl.ds(r0, nrows)])

            @pl.when(sub < NS - 1)
            def _():
                _flush(rows_t)

            @pl.when(sub == NS - 1)
            def _():
                _flush(last)

            plsc.subcore_barrier()

    return pl.kernel(
        body,
        out_type=[
            jax.ShapeDtypeStruct((NC, n, hh), F32),
            jax.ShapeDtypeStruct((3, NC, n, hh), F32),
        ],
        mesh=_sc_mesh(),
        compiler_params=pltpu.CompilerParams(use_tc_tiling_on_sc=False),
        scratch_types=[
            pltpu.VMEM((blk,), jnp.int32),
            pltpu.VMEM((blk,), jnp.int32),
            pltpu.VMEM((blk,), jnp.int32),
            pltpu.VMEM((blk, hh), F32),
            pltpu.VMEM((blk, hh), F32),
            pltpu.VMEM((blk, hh), F32),
            pltpu.VMEM((blk, hh), F32),
            pltpu.VMEM((blk, hh), F32),
            pltpu.VMEM((blk, hh), F32),
            pltpu.SemaphoreType.DMA,
            pltpu.SemaphoreType.DMA,
            pltpu.SemaphoreType.DMA,
            pltpu.SemaphoreType.DMA,
            pltpu.SemaphoreType.DMA,
            pltpu.VMEM_SHARED((n_pad, hh), F32),
        ],
    )


def _k2a_body(ns_ref, w1_ref, b1_ref, w2_ref, b2_ref, out_ref):
    x = ns_ref[...]
    y = lax.dot_general(x, w1_ref[...], (((1,), (1,)), ((), ())),
                        preferred_element_type=F32)
    hmid = _ssilu(y + b1_ref[...])
    z = lax.dot_general(hmid, w2_ref[...], (((1,), (1,)), ((), ())),
                        preferred_element_type=F32)
    z = z + b2_ref[...]
    hh = out_ref.shape[3]
    for t in range(3):
        for half in range(2):
            lo = t * 2 * hh + half * hh
            out_ref[t, half] = z[:, lo:lo + hh]


def _k2b_body(p4_ref, out_ref):
    p = p4_ref[0] + p4_ref[1]
    sums = p[:, 0:3]
    counts = p[:, 3:4]
    rv = sums / jnp.maximum(counts, 1.0)
    nrm = jnp.sqrt(jnp.sum(rv * rv, axis=-1, keepdims=True) + 1e-9)
    rv = rv / nrm
    col = lax.broadcasted_iota(jnp.int32, rv.shape, 1)
    default = jnp.where(col == 0, 1.0, 0.0).astype(rv.dtype)
    rv = jnp.where(nrm < 5e-5, default, rv)
    w = out_ref.shape[1]
    out_ref[...] = jnp.concatenate(
        [rv, jnp.zeros((rv.shape[0], w - 3), rv.dtype)], axis=1)


def _make_k4_body(e_tot, h, inv_sqrt_h):
    def body(rbf_ref, rvi_ref, ud_ref, we_ref, be_ref, wgt_ref, bg_ref,
             g_ref, bt_ref, out_ref):
        eb = pl.program_id(0)
        be_blk = rbf_ref.shape[0]
        ud = ud_ref[...]
        t = jnp.sum(ud * rvi_ref[:, 0:3], axis=-1, keepdims=True)
        p2 = 1.5 * t * t - 0.5
        mu = (1.0 + t + p2) / 3.0
        d0 = 1.0 - mu
        d1 = t - mu
        d2 = p2 - mu
        var = (d0 * d0 + d1 * d1 + d2 * d2) / 3.0
        inv = lax.rsqrt(var + 1e-5)
        ln0 = d0 * inv * g_ref[0, 0] + bt_ref[0, 0]
        ln1 = d1 * inv * g_ref[0, 1] + bt_ref[0, 1]
        ln2 = d2 * inv * g_ref[0, 2] + bt_ref[0, 2]
        u = (t * wgt_ref[0:1, :] + ln0 * wgt_ref[1:2, :]
             + ln1 * wgt_ref[2:3, :] + ln2 * wgt_ref[3:4, :] + bg_ref[...])
        gate = jnp.tanh(_ssilu(u))
        rbf_h = lax.dot_general(rbf_ref[...], we_ref[...],
                                (((1,), (1,)), ((), ())),
                                preferred_element_type=F32)
        rbf_h = rbf_h + be_ref[...]
        c = rbf_h * (1.0 + gate) * INV_SQRT_3
        row = eb * be_blk + lax.broadcasted_iota(jnp.int32, (be_blk, 1), 0)
        c = jnp.where(row < e_tot, c, 0.0)
        hh = h // 2
        c3 = c[:, 2 * h:3 * h]
        c1 = c[:, 0:h] * inv_sqrt_h
        c2 = c[:, h:2 * h] * inv_sqrt_h
        for half in range(2):
            lo = half * hh
            out_ref[0, half] = c3[:, lo:lo + hh]
            out_ref[1, half] = c1[:, lo:lo + hh]
            out_ref[2, half] = c2[:, lo:lo + hh] * ud[:, 0:1]
            out_ref[3, half] = c2[:, lo:lo + hh] * ud[:, 1:2]
            out_ref[4, half] = c2[:, lo:lo + hh] * ud[:, 2:3]
    return body


def _k6_body(p_ref, ds_ref, dv_ref):
    ds_ref[...] = p_ref[0, 0] + p_ref[0, 1]
    dv_ref[:, 0, :] = p_ref[1, 0] + p_ref[1, 1]
    dv_ref[:, 1, :] = p_ref[2, 0] + p_ref[2, 1]
    dv_ref[:, 2, :] = p_ref[3, 0] + p_ref[3, 1]


def kernel(node_scalar, node_vector, edge_index, edge_rbf, edge_udiff,
           W1, b1, W2, b2, We, be, gamma, beta, Wg, bg):
    n, h = node_scalar.shape
    e = edge_rbf.shape[0]
    blk = 128
    e_pad = ((e + NW * blk - 1) // (NW * blk)) * (NW * blk)
    n_pad = ((n + NS * 8 - 1) // (NS * 8)) * (NS * 8)

    w = 16
    jj = jnp.pad(edge_index[0], (0, e_pad - e)).astype(jnp.int32)
    ii = jnp.pad(edge_index[1], (0, e_pad - e)).astype(jnp.int32)
    ud4 = jnp.pad(
        jnp.concatenate([edge_udiff, jnp.ones((e, 1), F32)], axis=1),
        ((0, e_pad - e), (0, w - 4)))
    z4 = jnp.zeros((n_pad, w), F32)
    zh = jnp.zeros((n_pad, h // 2), F32)

    part4 = _make_k1(e_pad, n_pad, blk, w)(ii, ud4, z4)

    rv4 = pl.pallas_call(
        _k2b_body,
        grid=(1,),
        in_specs=[pl.BlockSpec((NC, n_pad, w), lambda b: (0, 0, 0))],
        out_specs=pl.BlockSpec((n_pad, w), lambda b: (0, 0)),
        out_shape=jax.ShapeDtypeStruct((n_pad, w), F32),
    )(part4)

    bn = 1000
    sp = pl.pallas_call(
        _k2a_body,
        grid=(n // bn,),
        in_specs=[
            pl.BlockSpec((bn, h), lambda b: (b, 0)),
            pl.BlockSpec(W1.shape, lambda b: (0, 0)),
            pl.BlockSpec((1, b1.shape[0]), lambda b: (0, 0)),
            pl.BlockSpec(W2.shape, lambda b: (0, 0)),
            pl.BlockSpec((1, b2.shape[0]), lambda b: (0, 0)),
        ],
        out_specs=pl.BlockSpec((3, 2, bn, h // 2), lambda b: (0, 0, b, 0)),
        out_shape=jax.ShapeDtypeStruct((3, 2, n, h // 2), F32),
    )(node_scalar, W1, b1.reshape(1, -1), W2, b2.reshape(1, -1))
    sp = sp.reshape(3, 2 * n, h // 2)

    rvi = _make_k3(e_pad, blk, w)(ii, rv4)

    be_blk = 1024
    rbf_p = jnp.pad(edge_rbf, ((0, e_pad - e), (0, 0)))
    ud_p = jnp.pad(edge_udiff, ((0, e_pad - e), (0, 0)))
    r = edge_rbf.shape[1]
    cs = pl.pallas_call(
        _make_k4_body(e, h, 1.0 / math.sqrt(h)),
        grid=(e_pad // be_blk,),
        in_specs=[
            pl.BlockSpec((be_blk, r), lambda b: (b, 0)),
            pl.BlockSpec((be_blk, w), lambda b: (b, 0)),
            pl.BlockSpec((be_blk, 3), lambda b: (b, 0)),
            pl.BlockSpec(We.shape, lambda b: (0, 0)),
            pl.BlockSpec((1, be.shape[0]), lambda b: (0, 0)),
            pl.BlockSpec((4, Wg.shape[0]), lambda b: (0, 0)),
            pl.BlockSpec((1, bg.shape[0]), lambda b: (0, 0)),
            pl.BlockSpec((1, 3), lambda b: (0, 0)),
            pl.BlockSpec((1, 3), lambda b: (0, 0)),
        ],
        out_specs=pl.BlockSpec((5, 2, be_blk, h // 2), lambda b: (0, 0, b, 0)),
        out_shape=jax.ShapeDtypeStruct((5, 2, e_pad, h // 2), F32),
    )(rbf_p, rvi, ud_p, We, be.reshape(1, -1), Wg.T, bg.reshape(1, -1),
      gamma.reshape(1, -1), beta.reshape(1, -1))
    cs = cs.reshape(5, 2 * e_pad, h // 2)

    nvc = jnp.transpose(node_vector, (1, 0, 2))
    nvt = jnp.concatenate([nvc[:, :, :h // 2], nvc[:, :, h // 2:]], axis=1)
    outs, outv = _make_k5(e_pad, n, n_pad, h, blk)(
        jj, ii, sp[0], sp[1], sp[2], nvt[0], nvt[1], nvt[2],
        cs[0], cs[1], cs[2], cs[3], cs[4], zh)
    ds = jnp.concatenate([outs[0], outs[1]], axis=-1)
    dv = jnp.transpose(
        jnp.concatenate([outv[:, 0], outv[:, 1]], axis=-1), (1, 0, 2))
    return ds, dv


# speedup vs baseline: 5.7832x; 1.1526x over previous
import functools
import math

import jax
import jax.numpy as jnp
from jax import lax
from jax.experimental import pallas as pl
from jax.experimental.pallas import tpu as pltpu
from jax.experimental.pallas import tpu_sc as plsc

F32 = jnp.float32
INV_SQRT_3 = 1.0 / math.sqrt(3.0)
SILU_SCALE = 1.0 / 0.6

NC = 2
NS = 16
NW = NC * NS

@functools.cache
def _sc_mesh():
    return plsc.VectorSubcoreMesh(
        core_axis_name="c", subcore_axis_name="s",
        num_cores=NC, num_subcores=NS)


@functools.cache
def _sc_mesh1():
    return plsc.VectorSubcoreMesh(
        core_axis_name="c", subcore_axis_name="s",
        num_cores=1, num_subcores=NS)


def _ssilu(x):
    return jax.nn.silu(x) * SILU_SCALE


def _make_k1(e_pad, n_pad, blk, w):
    ew = e_pad // NS
    nblk = ew // blk
    rows_t = n_pad // NS

    def body(idx_hbm, ud4_hbm, z4_hbm, out_hbm, idxv, udv, sem, acc4):
        sub = lax.axis_index("s")
        r0 = sub * rows_t
        pltpu.sync_copy(z4_hbm.at[pl.ds(r0, rows_t)], acc4.at[pl.ds(r0, rows_t)])
        plsc.subcore_barrier()
        wbase = sub * ew

        @pl.loop(0, nblk)
        def _(b):
            base = pl.multiple_of(wbase + b * blk, 8)
            pltpu.sync_copy(idx_hbm.at[pl.ds(base, blk)], idxv)
            pltpu.async_copy(ud4_hbm.at[pl.ds(base, blk)], udv, sem).wait()
            pltpu.sync_copy(udv, acc4.at[idxv], add=True)

        plsc.subcore_barrier()
        pltpu.sync_copy(acc4.at[pl.ds(r0, rows_t)],
                        out_hbm.at[pl.ds(r0, rows_t)])

    return pl.kernel(
        body,
        out_type=jax.ShapeDtypeStruct((n_pad, w), F32),
        mesh=_sc_mesh1(),
        compiler_params=pltpu.CompilerParams(use_tc_tiling_on_sc=False),
        scratch_types=[
            pltpu.VMEM((blk,), jnp.int32),
            pltpu.VMEM((blk, w), F32),
            pltpu.SemaphoreType.DMA,
            pltpu.VMEM_SHARED((n_pad, w), F32),
        ],
    )


def _make_k3(e_pad, blk, w):
    nblk = e_pad // NW // blk
    ew = e_pad // NW

    def body(idx_hbm, rv4_hbm, out_hbm, idxv, rvv, sem):
        core = lax.axis_index("c")
        sub = lax.axis_index("s")
        wbase = core * (e_pad // NC) + sub * ew

        @pl.loop(0, nblk)
        def _(b):
            base = pl.multiple_of(wbase + b * blk, 8)
            pltpu.sync_copy(idx_hbm.at[pl.ds(base, blk)], idxv)
            pltpu.async_copy(rv4_hbm.at[idxv], rvv, sem).wait()
            pltpu.sync_copy(rvv, out_hbm.at[pl.ds(base, blk)])

    return pl.kernel(
        body,
        out_type=jax.ShapeDtypeStruct((e_pad, w), F32),
        mesh=_sc_mesh(),
        compiler_params=pltpu.CompilerParams(use_tc_tiling_on_sc=False),
        scratch_types=[
            pltpu.VMEM((blk,), jnp.int32),
            pltpu.VMEM((blk, w), F32),
            pltpu.SemaphoreType.DMA,
        ],
    )


def _make_k5(e_pad, n, n_pad, h, blk):
    hh = h // 2
    ew = e_pad // NS
    nblk = ew // blk
    rows_t = n_pad // NS
    hv = hh // 16

    def body(idxj_hbm, idxi_hbm, t0, t1, t2, sp3, c3, cv0, cv1, cv2,
             z_hbm, outs_hbm, outv_hbm,
             ij0, ij1, ii0, ii1,
             gv0, gv1, cv0b, cv1b, av0, av1, dv0, dv1, ov0, ov1,
             sg0, sg1, acc):
        ij, iidx = (ij0, ij1), (ii0, ii1)
        gv, cvb = (gv0, gv1), (cv0b, cv1b)
        av, dvb, ov = (av0, av1), (dv0, dv1), (ov0, ov1)
        sems = ((sg0,) * 2, (sg1,) * 2)
        core = lax.axis_index("c")
        sub = lax.axis_index("s")
        r0 = sub * rows_t
        wbase = sub * ew
        last = n - (NS - 1) * rows_t
        joff = core * n
        eoff = core * e_pad

        chunks = ((sp3, c3), (t0, cv0), (t1, cv1), (t2, cv2))
        for q, (tab, coef) in enumerate(chunks):
            pltpu.sync_copy(z_hbm.at[pl.ds(r0, rows_t)],
                            acc.at[pl.ds(r0, rows_t)])
            plsc.subcore_barrier()

            def gsrcs(b, slot):
                cbase = pl.multiple_of(eoff + wbase + b * blk, 8)
                if q == 0:
                    return [(tab.at[ij[slot]], av[slot], sems[slot][0]),
                            (coef.at[pl.ds(cbase, blk)], dvb[slot],
                             sems[slot][1])]
                return [(tab.at[ij[slot]], gv[slot], sems[slot][0]),
                        (coef.at[pl.ds(cbase, blk)], cvb[slot],
                         sems[slot][1])]

            def issue(b, slot):
                base = pl.multiple_of(wbase + b * blk, 8)
                pltpu.sync_copy(idxj_hbm.at[pl.ds(base, blk)], ij[slot])
                pltpu.sync_copy(idxi_hbm.at[pl.ds(base, blk)], iidx[slot])

                @pl.loop(0, blk // 16)
                def _(k):
                    s = pl.ds(k * 16, 16)
                    ij[slot][s] = ij[slot][s] + joff

                for s_, d_, sm in gsrcs(b, slot):
                    pltpu.async_copy(s_, d_, sm)

            def process(b, slot):
                for s_, d_, sm in gsrcs(b, slot):
                    pltpu.make_async_copy(s_, d_, sm).wait()

                o_ = ov[slot]
                if q == 0:
                    a_, d_ = av[slot], dvb[slot]

                    @pl.loop(0, blk)
                    def _(r):
                        for cc in range(hv):
                            s = pl.ds(cc * 16, 16)
                            o_[r, s] = a_[r, s] * d_[r, s]
                else:
                    g_, c_ = gv[slot], cvb[slot]

                    @pl.loop(0, blk)
                    def _(r):
                        for cc in range(hv):
                            s = pl.ds(cc * 16, 16)
                            s2 = pl.ds(hh + cc * 16, 16)
                            s3 = pl.ds(2 * hh + cc * 16, 16)
                            o_[r, s] = (g_[r, s] * c_[r, s] * g_[r, s3]
                                        + g_[r, s2] * c_[r, s2])

                pltpu.sync_copy(o_, acc.at[iidx[slot]], add=True)

            issue(0, 0)

            @pl.loop(0, nblk, step=2)
            def _(b):
                issue(b + 1, 1)
                process(b, 0)

                @pl.when(b + 2 < nblk)
                def _():
                    issue(b + 2, 0)

                process(b + 1, 1)

            plsc.subcore_barrier()

            def _flush(nrows):
                if q == 0:
                    pltpu.sync_copy(acc.at[pl.ds(r0, nrows)],
                                    outs_hbm.at[core, pl.ds(r0, nrows)])
                else:
                    pltpu.sync_copy(acc.at[pl.ds(r0, nrows)],
                                    outv_hbm.at[q - 1, core,
                                                pl.ds(r0, nrows)])

            @pl.when(sub < NS - 1)
            def _():
                _flush(rows_t)

            @pl.when(sub == NS - 1)
            def _():
                _flush(last)

            plsc.subcore_barrier()

    dma = pltpu.SemaphoreType.DMA
    return pl.kernel(
        body,
        out_type=[
            jax.ShapeDtypeStruct((2, n, hh), F32),
            jax.ShapeDtypeStruct((3, 2, n, hh), F32),
        ],
        mesh=_sc_mesh(),
        compiler_params=pltpu.CompilerParams(use_tc_tiling_on_sc=False),
        scratch_types=(
            [pltpu.VMEM((blk,), jnp.int32)] * 4
            + [pltpu.VMEM((blk, 3 * hh), F32)] * 2
            + [pltpu.VMEM((blk, 2 * hh), F32)] * 2
            + [pltpu.VMEM((blk, hh), F32)] * 6
            + [dma] * 2
            + [pltpu.VMEM_SHARED((n_pad, hh), F32)]
        ),
    )


def _k2a_body(ns_ref, w1_ref, b1_ref, w2_ref, b2_ref, out_ref):
    x = ns_ref[...]
    y = lax.dot_general(x, w1_ref[...], (((1,), (1,)), ((), ())),
                        preferred_element_type=F32)
    hmid = _ssilu(y + b1_ref[...])
    z = lax.dot_general(hmid, w2_ref[...], (((1,), (1,)), ((), ())),
                        preferred_element_type=F32)
    z = z + b2_ref[...]
    hh = out_ref.shape[3]
    for t in range(3):
        for half in range(2):
            lo = t * 2 * hh + half * hh
            out_ref[t, half] = z[:, lo:lo + hh]


def _k2b_body(p4_ref, out_ref):
    p = p4_ref[...]
    sums = p[:, 0:3]
    counts = p[:, 3:4]
    rv = sums / jnp.maximum(counts, 1.0)
    nrm = jnp.sqrt(jnp.sum(rv * rv, axis=-1, keepdims=True) + 1e-9)
    rv = rv / nrm
    col = lax.broadcasted_iota(jnp.int32, rv.shape, 1)
    default = jnp.where(col == 0, 1.0, 0.0).astype(rv.dtype)
    rv = jnp.where(nrm < 5e-5, default, rv)
    w = out_ref.shape[1]
    out_ref[...] = jnp.concatenate(
        [rv, jnp.zeros((rv.shape[0], w - 3), rv.dtype)], axis=1)


def _make_k4_body(e_tot, h, inv_sqrt_h):
    def body(rbf_ref, rvi_ref, ud_ref, we_ref, be_ref, wgt_ref, bg_ref,
             g_ref, bt_ref, cs_ref, cv_ref):
        eb = pl.program_id(0)
        be_blk = rbf_ref.shape[0]
        ud = ud_ref[...]
        t = jnp.sum(ud * rvi_ref[:, 0:3], axis=-1, keepdims=True)
        p2 = 1.5 * t * t - 0.5
        mu = (1.0 + t + p2) / 3.0
        d0 = 1.0 - mu
        d1 = t - mu
        d2 = p2 - mu
        var = (d0 * d0 + d1 * d1 + d2 * d2) / 3.0
        inv = lax.rsqrt(var + 1e-5)
        ln0 = d0 * inv * g_ref[0, 0] + bt_ref[0, 0]
        ln1 = d1 * inv * g_ref[0, 1] + bt_ref[0, 1]
        ln2 = d2 * inv * g_ref[0, 2] + bt_ref[0, 2]
        u = (t * wgt_ref[0:1, :] + ln0 * wgt_ref[1:2, :]
             + ln1 * wgt_ref[2:3, :] + ln2 * wgt_ref[3:4, :] + bg_ref[...])
        gate = jnp.tanh(_ssilu(u))
        rbf_h = lax.dot_general(rbf_ref[...], we_ref[...],
                                (((1,), (1,)), ((), ())),
                                preferred_element_type=F32)
        rbf_h = rbf_h + be_ref[...]
        c = rbf_h * (1.0 + gate) * INV_SQRT_3
        row = eb * be_blk + lax.broadcasted_iota(jnp.int32, (be_blk, 1), 0)
        c = jnp.where(row < e_tot, c, 0.0)
        hh = h // 2
        c3 = c[:, 2 * h:3 * h]
        c1 = c[:, 0:h] * inv_sqrt_h
        c2 = c[:, h:2 * h] * inv_sqrt_h
        for half in range(2):
            lo = half * hh
            cs_ref[half] = c3[:, lo:lo + hh]
            for d in range(3):
                cv_ref[d, half, :, 0:hh] = c1[:, lo:lo + hh]
                cv_ref[d, half, :, hh:2 * hh] = (c2[:, lo:lo + hh]
                                                 * ud[:, d:d + 1])
    return body


def _k6_body(p_ref, ds_ref, dv_ref):
    ds_ref[...] = p_ref[0, 0] + p_ref[0, 1]
    dv_ref[:, 0, :] = p_ref[1, 0] + p_ref[1, 1]
    dv_ref[:, 1, :] = p_ref[2, 0] + p_ref[2, 1]
    dv_ref[:, 2, :] = p_ref[3, 0] + p_ref[3, 1]


def kernel(node_scalar, node_vector, edge_index, edge_rbf, edge_udiff,
           W1, b1, W2, b2, We, be, gamma, beta, Wg, bg):
    n, h = node_scalar.shape
    e = edge_rbf.shape[0]
    blk = 128
    e_pad = ((e + NW * blk - 1) // (NW * blk)) * (NW * blk)
    n_pad = ((n + NS * 8 - 1) // (NS * 8)) * (NS * 8)

    w = 16
    jj = jnp.pad(edge_index[0], (0, e_pad - e)).astype(jnp.int32)
    ii = jnp.pad(edge_index[1], (0, e_pad - e)).astype(jnp.int32)
    ud4 = jnp.pad(
        jnp.concatenate([edge_udiff, jnp.ones((e, 1), F32)], axis=1),
        ((0, e_pad - e), (0, w - 4)))
    z4 = jnp.zeros((n_pad, w), F32)
    zh = jnp.zeros((n_pad, h // 2), F32)
    blk5 = 80

    part4 = _make_k1(e_pad, n_pad, blk, w)(ii, ud4, z4)

    rv4 = pl.pallas_call(
        _k2b_body,
        grid=(1,),
        in_specs=[pl.BlockSpec((n_pad, w), lambda b: (0, 0))],
        out_specs=pl.BlockSpec((n_pad, w), lambda b: (0, 0)),
        out_shape=jax.ShapeDtypeStruct((n_pad, w), F32),
    )(part4)

    bn = 1000
    sp = pl.pallas_call(
        _k2a_body,
        grid=(n // bn,),
        in_specs=[
            pl.BlockSpec((bn, h), lambda b: (b, 0)),
            pl.BlockSpec(W1.shape, lambda b: (0, 0)),
            pl.BlockSpec((1, b1.shape[0]), lambda b: (0, 0)),
            pl.BlockSpec(W2.shape, lambda b: (0, 0)),
            pl.BlockSpec((1, b2.shape[0]), lambda b: (0, 0)),
        ],
        out_specs=pl.BlockSpec((3, 2, bn, h // 2), lambda b: (0, 0, b, 0)),
        out_shape=jax.ShapeDtypeStruct((3, 2, n, h // 2), F32),
    )(node_scalar, W1, b1.reshape(1, -1), W2, b2.reshape(1, -1))
    sp = sp.reshape(3, 2 * n, h // 2)

    rvi = _make_k3(e_pad, blk, w)(ii, rv4)

    be_blk = 1024
    rbf_p = jnp.pad(edge_rbf, ((0, e_pad - e), (0, 0)))
    ud_p = jnp.pad(edge_udiff, ((0, e_pad - e), (0, 0)))
    r = edge_rbf.shape[1]
    cs = pl.pallas_call(
        _make_k4_body(e, h, 1.0 / math.sqrt(h)),
        grid=(e_pad // be_blk,),
        in_specs=[
            pl.BlockSpec((be_blk, r), lambda b: (b, 0)),
            pl.BlockSpec((be_blk, w), lambda b: (b, 0)),
            pl.BlockSpec((be_blk, 3), lambda b: (b, 0)),
            pl.BlockSpec(We.shape, lambda b: (0, 0)),
            pl.BlockSpec((1, be.shape[0]), lambda b: (0, 0)),
            pl.BlockSpec((4, Wg.shape[0]), lambda b: (0, 0)),
            pl.BlockSpec((1, bg.shape[0]), lambda b: (0, 0)),
            pl.BlockSpec((1, 3), lambda b: (0, 0)),
            pl.BlockSpec((1, 3), lambda b: (0, 0)),
        ],
        out_specs=[
            pl.BlockSpec((2, be_blk, h // 2), lambda b: (0, b, 0)),
            pl.BlockSpec((3, 2, be_blk, h), lambda b: (0, 0, b, 0)),
        ],
        out_shape=[
            jax.ShapeDtypeStruct((2, e_pad, h // 2), F32),
            jax.ShapeDtypeStruct((3, 2, e_pad, h), F32),
        ],
    )(rbf_p, rvi, ud_p, We, be.reshape(1, -1), Wg.T, bg.reshape(1, -1),
      gamma.reshape(1, -1), beta.reshape(1, -1))
    cs3 = cs[0].reshape(2 * e_pad, h // 2)
    cvt = cs[1].reshape(3, 2 * e_pad, h)

    nvc = jnp.transpose(node_vector, (1, 0, 2))
    nvt = jnp.concatenate([nvc[:, :, :h // 2], nvc[:, :, h // 2:]], axis=1)
    tcat = jnp.stack([
        jnp.concatenate([sp[0], sp[1], nvt[d]], axis=1) for d in range(3)])
    outs, outv = _make_k5(e_pad, n, n_pad, h, blk5)(
        jj, ii, tcat[0], tcat[1], tcat[2], sp[2],
        cs3, cvt[0], cvt[1], cvt[2], zh)
    ds = jnp.concatenate([outs[0], outs[1]], axis=-1)
    dv = jnp.transpose(
        jnp.concatenate([outv[:, 0], outv[:, 1]], axis=-1), (1, 0, 2))
    return ds, dv


# speedup vs baseline: 5.9553x; 1.0297x over previous
import functools
import math

import jax
import jax.numpy as jnp
from jax import lax
from jax.experimental import pallas as pl
from jax.experimental.pallas import tpu as pltpu
from jax.experimental.pallas import tpu_sc as plsc

F32 = jnp.float32
INV_SQRT_3 = 1.0 / math.sqrt(3.0)
SILU_SCALE = 1.0 / 0.6

NC = 2
NS = 16
NW = NC * NS

@functools.cache
def _sc_mesh():
    return plsc.VectorSubcoreMesh(
        core_axis_name="c", subcore_axis_name="s",
        num_cores=NC, num_subcores=NS)


@functools.cache
def _sc_mesh1():
    return plsc.VectorSubcoreMesh(
        core_axis_name="c", subcore_axis_name="s",
        num_cores=1, num_subcores=NS)


def _ssilu(x):
    return jax.nn.silu(x) * SILU_SCALE


def _make_k1(e_pad, n_pad, blk, w):
    ew = e_pad // NS
    nblk = ew // blk
    rows_t = n_pad // NS

    def body(idx_hbm, ud4_hbm, z4_hbm, out_hbm, idxv, udv, sem, acc4):
        sub = lax.axis_index("s")
        r0 = sub * rows_t
        pltpu.sync_copy(z4_hbm.at[pl.ds(r0, rows_t)], acc4.at[pl.ds(r0, rows_t)])
        plsc.subcore_barrier()
        wbase = sub * ew

        @pl.loop(0, nblk)
        def _(b):
            base = pl.multiple_of(wbase + b * blk, 8)
            pltpu.sync_copy(idx_hbm.at[pl.ds(base, blk)], idxv)
            pltpu.async_copy(ud4_hbm.at[pl.ds(base, blk)], udv, sem).wait()
            pltpu.sync_copy(udv, acc4.at[idxv], add=True)

        plsc.subcore_barrier()
        pltpu.sync_copy(acc4.at[pl.ds(r0, rows_t)],
                        out_hbm.at[pl.ds(r0, rows_t)])

    return pl.kernel(
        body,
        out_type=jax.ShapeDtypeStruct((n_pad, w), F32),
        mesh=_sc_mesh1(),
        compiler_params=pltpu.CompilerParams(use_tc_tiling_on_sc=False),
        scratch_types=[
            pltpu.VMEM((blk,), jnp.int32),
            pltpu.VMEM((blk, w), F32),
            pltpu.SemaphoreType.DMA,
            pltpu.VMEM_SHARED((n_pad, w), F32),
        ],
    )


def _make_k3(e_pad, blk, w):
    nblk = e_pad // NW // blk
    ew = e_pad // NW

    def body(idx_hbm, rv4_hbm, out_hbm, idxv, rvv, sem):
        core = lax.axis_index("c")
        sub = lax.axis_index("s")
        wbase = core * (e_pad // NC) + sub * ew

        @pl.loop(0, nblk)
        def _(b):
            base = pl.multiple_of(wbase + b * blk, 8)
            pltpu.sync_copy(idx_hbm.at[pl.ds(base, blk)], idxv)
            pltpu.async_copy(rv4_hbm.at[idxv], rvv, sem).wait()
            pltpu.sync_copy(rvv, out_hbm.at[pl.ds(base, blk)])

    return pl.kernel(
        body,
        out_type=jax.ShapeDtypeStruct((e_pad, w), F32),
        mesh=_sc_mesh(),
        compiler_params=pltpu.CompilerParams(use_tc_tiling_on_sc=False),
        scratch_types=[
            pltpu.VMEM((blk,), jnp.int32),
            pltpu.VMEM((blk, w), F32),
            pltpu.SemaphoreType.DMA,
        ],
    )


def _make_k5(e_pad, n, n_pad, h, blk):
    hh = h // 2
    ew = e_pad // NS
    nblk = ew // blk
    rows_t = n_pad // NS
    hv = hh // 16

    def body(idx_hbm, t0, t1, t2, sp3, c3, cv0, cv1, cv2,
             z_hbm, outs_hbm, outv_hbm,
             ji0, ji1,
             gv0, gv1, cv0b, cv1b, av0, av1, dv0, dv1, ov0, ov1,
             sg0, sg1, acc):
        ji = (ji0, ji1)
        gv, cvb = (gv0, gv1), (cv0b, cv1b)
        av, dvb, ov = (av0, av1), (dv0, dv1), (ov0, ov1)
        sems = ((sg0,) * 2, (sg1,) * 2)
        core = lax.axis_index("c")
        sub = lax.axis_index("s")
        r0 = sub * rows_t
        wbase = sub * ew
        last = n - (NS - 1) * rows_t
        joff = core * n
        eoff = core * e_pad

        chunks = ((sp3, c3), (t0, cv0), (t1, cv1), (t2, cv2))
        for q, (tab, coef) in enumerate(chunks):
            pltpu.sync_copy(z_hbm.at[pl.ds(r0, rows_t)],
                            acc.at[pl.ds(r0, rows_t)])
            plsc.subcore_barrier()

            def gsrcs(b, slot):
                cbase = pl.multiple_of(eoff + wbase + b * blk, 8)
                if q == 0:
                    return [(tab.at[ji[slot].at[0]], av[slot],
                             sems[slot][0]),
                            (coef.at[pl.ds(cbase, blk)], dvb[slot],
                             sems[slot][1])]
                return [(tab.at[ji[slot].at[0]], gv[slot], sems[slot][0]),
                        (coef.at[pl.ds(cbase, blk)], cvb[slot],
                         sems[slot][1])]

            def issue(b, slot):
                gb = sub * nblk + b
                pltpu.sync_copy(idx_hbm.at[pl.ds(2 * gb, 2)], ji[slot])

                @pl.loop(0, blk // 16)
                def _(k):
                    s = pl.ds(k * 16, 16)
                    ji[slot][0, s] = ji[slot][0, s] + joff

                for s_, d_, sm in gsrcs(b, slot):
                    pltpu.async_copy(s_, d_, sm)

            def process(b, slot):
                for s_, d_, sm in gsrcs(b, slot):
                    pltpu.make_async_copy(s_, d_, sm).wait()

                o_ = ov[slot]
                if q == 0:
                    a_, d_ = av[slot], dvb[slot]

                    @pl.loop(0, blk)
                    def _(r):
                        for cc in range(hv):
                            s = pl.ds(cc * 16, 16)
                            o_[r, s] = a_[r, s] * d_[r, s]
                else:
                    g_, c_ = gv[slot], cvb[slot]

                    @pl.loop(0, blk)
                    def _(r):
                        for cc in range(hv):
                            s = pl.ds(cc * 16, 16)
                            s2 = pl.ds(hh + cc * 16, 16)
                            s3 = pl.ds(2 * hh + cc * 16, 16)
                            o_[r, s] = (g_[r, s] * c_[r, s] * g_[r, s3]
                                        + g_[r, s2] * c_[r, s2])

                pltpu.sync_copy(o_, acc.at[ji[slot].at[1]], add=True)

            issue(0, 0)

            @pl.loop(0, nblk, step=2)
            def _(b):
                issue(b + 1, 1)
                process(b, 0)

                @pl.when(b + 2 < nblk)
                def _():
                    issue(b + 2, 0)

                process(b + 1, 1)

            plsc.subcore_barrier()

            def _flush(nrows):
                if q == 0:
                    pltpu.sync_copy(acc.at[pl.ds(r0, nrows)],
                                    outs_hbm.at[core, pl.ds(r0, nrows)])
                else:
                    pltpu.sync_copy(acc.at[pl.ds(r0, nrows)],
                                    outv_hbm.at[q - 1, core,
                                                pl.ds(r0, nrows)])

            @pl.when(sub < NS - 1)
            def _():
                _flush(rows_t)

            @pl.when(sub == NS - 1)
            def _():
                _flush(last)

            plsc.subcore_barrier()

    dma = pltpu.SemaphoreType.DMA
    return pl.kernel(
        body,
        out_type=[
            jax.ShapeDtypeStruct((2, n, hh), F32),
            jax.ShapeDtypeStruct((3, 2, n, hh), F32),
        ],
        mesh=_sc_mesh(),
        compiler_params=pltpu.CompilerParams(use_tc_tiling_on_sc=False),
        scratch_types=(
            [pltpu.VMEM((2, blk), jnp.int32)] * 2
            + [pltpu.VMEM((blk, 3 * hh), F32)] * 2
            + [pltpu.VMEM((blk, 2 * hh), F32)] * 2
            + [pltpu.VMEM((blk, hh), F32)] * 6
            + [dma] * 2
            + [pltpu.VMEM_SHARED((n_pad, hh), F32)]
        ),
    )


def _k2a_body(ns_ref, w1_ref, b1_ref, w2_ref, b2_ref, out_ref):
    x = ns_ref[...]
    y = lax.dot_general(x, w1_ref[...], (((1,), (1,)), ((), ())),
                        preferred_element_type=F32)
    hmid = _ssilu(y + b1_ref[...])
    z = lax.dot_general(hmid, w2_ref[...], (((1,), (1,)), ((), ())),
                        preferred_element_type=F32)
    z = z + b2_ref[...]
    hh = out_ref.shape[3]
    for t in range(3):
        for half in range(2):
            lo = t * 2 * hh + half * hh
            out_ref[t, half] = z[:, lo:lo + hh]


def _k2b_body(p4_ref, out_ref):
    p = p4_ref[...]
    sums = p[:, 0:3]
    counts = p[:, 3:4]
    rv = sums / jnp.maximum(counts, 1.0)
    nrm = jnp.sqrt(jnp.sum(rv * rv, axis=-1, keepdims=True) + 1e-9)
    rv = rv / nrm
    col = lax.broadcasted_iota(jnp.int32, rv.shape, 1)
    default = jnp.where(col == 0, 1.0, 0.0).astype(rv.dtype)
    rv = jnp.where(nrm < 5e-5, default, rv)
    w = out_ref.shape[1]
    out_ref[...] = jnp.concatenate(
        [rv, jnp.zeros((rv.shape[0], w - 3), rv.dtype)], axis=1)


def _make_k4_body(e_tot, h, inv_sqrt_h):
    def body(rbf_ref, rvi_ref, ud_ref, we_ref, be_ref, wgt_ref, bg_ref,
             g_ref, bt_ref, cs_ref, cv_ref):
        eb = pl.program_id(0)
        be_blk = rbf_ref.shape[0]
        ud = ud_ref[...]
        t = jnp.sum(ud * rvi_ref[:, 0:3], axis=-1, keepdims=True)
        p2 = 1.5 * t * t - 0.5
        mu = (1.0 + t + p2) / 3.0
        d0 = 1.0 - mu
        d1 = t - mu
        d2 = p2 - mu
        var = (d0 * d0 + d1 * d1 + d2 * d2) / 3.0
        inv = lax.rsqrt(var + 1e-5)
        ln0 = d0 * inv * g_ref[0, 0] + bt_ref[0, 0]
        ln1 = d1 * inv * g_ref[0, 1] + bt_ref[0, 1]
        ln2 = d2 * inv * g_ref[0, 2] + bt_ref[0, 2]
        u = (t * wgt_ref[0:1, :] + ln0 * wgt_ref[1:2, :]
             + ln1 * wgt_ref[2:3, :] + ln2 * wgt_ref[3:4, :] + bg_ref[...])
        gate = jnp.tanh(_ssilu(u))
        rbf_h = lax.dot_general(rbf_ref[...], we_ref[...],
                                (((1,), (1,)), ((), ())),
                                preferred_element_type=F32)
        rbf_h = rbf_h + be_ref[...]
        c = rbf_h * (1.0 + gate) * INV_SQRT_3
        row = eb * be_blk + lax.broadcasted_iota(jnp.int32, (be_blk, 1), 0)
        c = jnp.where(row < e_tot, c, 0.0)
        hh = h // 2
        c3 = c[:, 2 * h:3 * h]
        c1 = c[:, 0:h] * inv_sqrt_h
        c2 = c[:, h:2 * h] * inv_sqrt_h
        for half in range(2):
            lo = half * hh
            cs_ref[half] = c3[:, lo:lo + hh]
            for d in range(3):
                cv_ref[d, half, :, 0:hh] = c1[:, lo:lo + hh]
                cv_ref[d, half, :, hh:2 * hh] = (c2[:, lo:lo + hh]
                                                 * ud[:, d:d + 1])
    return body


def _k6_body(p_ref, ds_ref, dv_ref):
    ds_ref[...] = p_ref[0, 0] + p_ref[0, 1]
    dv_ref[:, 0, :] = p_ref[1, 0] + p_ref[1, 1]
    dv_ref[:, 1, :] = p_ref[2, 0] + p_ref[2, 1]
    dv_ref[:, 2, :] = p_ref[3, 0] + p_ref[3, 1]


def kernel(node_scalar, node_vector, edge_index, edge_rbf, edge_udiff,
           W1, b1, W2, b2, We, be, gamma, beta, Wg, bg):
    n, h = node_scalar.shape
    e = edge_rbf.shape[0]
    blk = 128
    e_pad = ((e + NW * blk - 1) // (NW * blk)) * (NW * blk)
    n_pad = ((n + NS * 8 - 1) // (NS * 8)) * (NS * 8)

    w = 16
    jj = jnp.pad(edge_index[0], (0, e_pad - e)).astype(jnp.int32)
    ii = jnp.pad(edge_index[1], (0, e_pad - e)).astype(jnp.int32)
    ud4 = jnp.pad(
        jnp.concatenate([edge_udiff, jnp.ones((e, 1), F32)], axis=1),
        ((0, e_pad - e), (0, w - 4)))
    z4 = jnp.zeros((n_pad, w), F32)
    zh = jnp.zeros((n_pad, h // 2), F32)
    blk5 = 80

    part4 = _make_k1(e_pad, n_pad, blk, w)(ii, ud4, z4)

    rv4 = pl.pallas_call(
        _k2b_body,
        grid=(1,),
        in_specs=[pl.BlockSpec((n_pad, w), lambda b: (0, 0))],
        out_specs=pl.BlockSpec((n_pad, w), lambda b: (0, 0)),
        out_shape=jax.ShapeDtypeStruct((n_pad, w), F32),
    )(part4)

    bn = 1000
    sp = pl.pallas_call(
        _k2a_body,
        grid=(n // bn,),
        in_specs=[
            pl.BlockSpec((bn, h), lambda b: (b, 0)),
            pl.BlockSpec(W1.shape, lambda b: (0, 0)),
            pl.BlockSpec((1, b1.shape[0]), lambda b: (0, 0)),
            pl.BlockSpec(W2.shape, lambda b: (0, 0)),
            pl.BlockSpec((1, b2.shape[0]), lambda b: (0, 0)),
        ],
        out_specs=pl.BlockSpec((3, 2, bn, h // 2), lambda b: (0, 0, b, 0)),
        out_shape=jax.ShapeDtypeStruct((3, 2, n, h // 2), F32),
    )(node_scalar, W1, b1.reshape(1, -1), W2, b2.reshape(1, -1))
    sp = sp.reshape(3, 2 * n, h // 2)

    rvi = _make_k3(e_pad, blk, w)(ii, rv4)

    be_blk = 1024
    rbf_p = jnp.pad(edge_rbf, ((0, e_pad - e), (0, 0)))
    ud_p = jnp.pad(edge_udiff, ((0, e_pad - e), (0, 0)))
    r = edge_rbf.shape[1]
    cs = pl.pallas_call(
        _make_k4_body(e, h, 1.0 / math.sqrt(h)),
        grid=(e_pad // be_blk,),
        in_specs=[
            pl.BlockSpec((be_blk, r), lambda b: (b, 0)),
            pl.BlockSpec((be_blk, w), lambda b: (b, 0)),
            pl.BlockSpec((be_blk, 3), lambda b: (b, 0)),
            pl.BlockSpec(We.shape, lambda b: (0, 0)),
            pl.BlockSpec((1, be.shape[0]), lambda b: (0, 0)),
            pl.BlockSpec((4, Wg.shape[0]), lambda b: (0, 0)),
            pl.BlockSpec((1, bg.shape[0]), lambda b: (0, 0)),
            pl.BlockSpec((1, 3), lambda b: (0, 0)),
            pl.BlockSpec((1, 3), lambda b: (0, 0)),
        ],
        out_specs=[
            pl.BlockSpec((2, be_blk, h // 2), lambda b: (0, b, 0)),
            pl.BlockSpec((3, 2, be_blk, h), lambda b: (0, 0, b, 0)),
        ],
        out_shape=[
            jax.ShapeDtypeStruct((2, e_pad, h // 2), F32),
            jax.ShapeDtypeStruct((3, 2, e_pad, h), F32),
        ],
    )(rbf_p, rvi, ud_p, We, be.reshape(1, -1), Wg.T, bg.reshape(1, -1),
      gamma.reshape(1, -1), beta.reshape(1, -1))
    cs3 = cs[0].reshape(2 * e_pad, h // 2)
    cvt = cs[1].reshape(3, 2 * e_pad, h)

    nvc = jnp.transpose(node_vector, (1, 0, 2))
    nvt = jnp.concatenate([nvc[:, :, :h // 2], nvc[:, :, h // 2:]], axis=1)
    tcat = jnp.stack([
        jnp.concatenate([sp[0], sp[1], nvt[d]], axis=1) for d in range(3)])
    ji = jnp.stack([jj.reshape(e_pad // blk5, blk5),
                    ii.reshape(e_pad // blk5, blk5)],
                   axis=1).reshape(2 * e_pad // blk5, blk5)
    outs, outv = _make_k5(e_pad, n, n_pad, h, blk5)(
        ji, tcat[0], tcat[1], tcat[2], sp[2],
        cs3, cvt[0], cvt[1], cvt[2], zh)
    ds = jnp.concatenate([outs[0], outs[1]], axis=-1)
    dv = jnp.transpose(
        jnp.concatenate([outv[:, 0], outv[:, 1]], axis=-1), (1, 0, 2))
    return ds, dv


# speedup vs baseline: 6.0749x; 1.0201x over previous
import functools
import math

import jax
import jax.numpy as jnp
from jax import lax
from jax.experimental import pallas as pl
from jax.experimental.pallas import tpu as pltpu
from jax.experimental.pallas import tpu_sc as plsc

F32 = jnp.float32
INV_SQRT_3 = 1.0 / math.sqrt(3.0)
SILU_SCALE = 1.0 / 0.6

NC = 2
NS = 16
NW = NC * NS

@functools.cache
def _sc_mesh():
    return plsc.VectorSubcoreMesh(
        core_axis_name="c", subcore_axis_name="s",
        num_cores=NC, num_subcores=NS)


@functools.cache
def _sc_mesh1():
    return plsc.VectorSubcoreMesh(
        core_axis_name="c", subcore_axis_name="s",
        num_cores=1, num_subcores=NS)


def _ssilu(x):
    return jax.nn.silu(x) * SILU_SCALE


def _make_k1(e_pad, n_pad, blk, w):
    ew = e_pad // NS
    nblk = ew // blk
    rows_t = n_pad // NS

    def body(idx_hbm, ud4_hbm, z4_hbm, out_hbm, idxv, udv, sem, acc4):
        sub = lax.axis_index("s")
        r0 = sub * rows_t
        pltpu.sync_copy(z4_hbm.at[pl.ds(r0, rows_t)], acc4.at[pl.ds(r0, rows_t)])
        plsc.subcore_barrier()
        wbase = sub * ew

        @pl.loop(0, nblk)
        def _(b):
            base = pl.multiple_of(wbase + b * blk, 8)
            pltpu.sync_copy(idx_hbm.at[pl.ds(base, blk)], idxv)
            pltpu.async_copy(ud4_hbm.at[pl.ds(base, blk)], udv, sem).wait()
            pltpu.sync_copy(udv, acc4.at[idxv], add=True)

        plsc.subcore_barrier()
        pltpu.sync_copy(acc4.at[pl.ds(r0, rows_t)],
                        out_hbm.at[pl.ds(r0, rows_t)])

    return pl.kernel(
        body,
        out_type=jax.ShapeDtypeStruct((n_pad, w), F32),
        mesh=_sc_mesh1(),
        compiler_params=pltpu.CompilerParams(use_tc_tiling_on_sc=False),
        scratch_types=[
            pltpu.VMEM((blk,), jnp.int32),
            pltpu.VMEM((blk, w), F32),
            pltpu.SemaphoreType.DMA,
            pltpu.VMEM_SHARED((n_pad, w), F32),
        ],
    )


def _make_k3(e_pad, blk, w):
    nblk = e_pad // NW // blk
    ew = e_pad // NW

    def body(idx_hbm, rv4_hbm, out_hbm, idxv, rvv, sem):
        core = lax.axis_index("c")
        sub = lax.axis_index("s")
        wbase = core * (e_pad // NC) + sub * ew

        @pl.loop(0, nblk)
        def _(b):
            base = pl.multiple_of(wbase + b * blk, 8)
            pltpu.sync_copy(idx_hbm.at[pl.ds(base, blk)], idxv)
            pltpu.async_copy(rv4_hbm.at[idxv], rvv, sem).wait()
            pltpu.sync_copy(rvv, out_hbm.at[pl.ds(base, blk)])

    return pl.kernel(
        body,
        out_type=jax.ShapeDtypeStruct((e_pad, w), F32),
        mesh=_sc_mesh(),
        compiler_params=pltpu.CompilerParams(use_tc_tiling_on_sc=False),
        scratch_types=[
            pltpu.VMEM((blk,), jnp.int32),
            pltpu.VMEM((blk, w), F32),
            pltpu.SemaphoreType.DMA,
        ],
    )


def _make_k5(e_pad, n, n_pad, h, blk):
    hh = h // 2
    ew = e_pad // NS
    nblk = ew // blk
    rows_t = n_pad // NS
    hv = hh // 16

    def body(idx_hbm, t0, t1, t2, sp3, c3, cv0, cv1, cv2,
             z_hbm, outs_hbm, outv_hbm,
             ji0, ji1, is0, is1,
             gv0, gv1, cv0b, cv1b, av0, av1, dv0, dv1, ov0, ov1,
             sg0, sg1, ssc0, ssc1, acc):
        ji = (ji0, ji1)
        isc = (is0, is1)
        ssc = (ssc0, ssc1)
        gv, cvb = (gv0, gv1), (cv0b, cv1b)
        av, dvb, ov = (av0, av1), (dv0, dv1), (ov0, ov1)
        sems = ((sg0,) * 2, (sg1,) * 2)
        core = lax.axis_index("c")
        sub = lax.axis_index("s")
        r0 = sub * rows_t
        wbase = sub * ew
        last = n - (NS - 1) * rows_t
        joff = core * n
        eoff = core * e_pad

        chunks = ((sp3, c3), (t0, cv0), (t1, cv1), (t2, cv2))
        for q, (tab, coef) in enumerate(chunks):
            pltpu.sync_copy(z_hbm.at[pl.ds(r0, rows_t)],
                            acc.at[pl.ds(r0, rows_t)])
            plsc.subcore_barrier()

            def gsrcs(b, slot):
                cbase = pl.multiple_of(eoff + wbase + b * blk, 8)
                if q == 0:
                    return [(tab.at[ji[slot].at[0]], av[slot],
                             sems[slot][0]),
                            (coef.at[pl.ds(cbase, blk)], dvb[slot],
                             sems[slot][1])]
                return [(tab.at[ji[slot].at[0]], gv[slot], sems[slot][0]),
                        (coef.at[pl.ds(cbase, blk)], cvb[slot],
                         sems[slot][1])]

            def issue(b, slot):
                gb = sub * nblk + b
                pltpu.sync_copy(idx_hbm.at[pl.ds(2 * gb, 2)], ji[slot])

                @pl.loop(0, blk // 16)
                def _(k):
                    s = pl.ds(k * 16, 16)
                    ji[slot][0, s] = ji[slot][0, s] + joff

                for s_, d_, sm in gsrcs(b, slot):
                    pltpu.async_copy(s_, d_, sm)

            def process(b, slot):
                for s_, d_, sm in gsrcs(b, slot):
                    pltpu.make_async_copy(s_, d_, sm).wait()

                @pl.when(b >= 2)
                def _():
                    pltpu.make_async_copy(ov[slot], acc.at[isc[slot]],
                                          ssc[slot]).wait()

                o_ = ov[slot]
                if q == 0:
                    a_, d_ = av[slot], dvb[slot]

                    @pl.loop(0, blk)
                    def _(r):
                        for cc in range(hv):
                            s = pl.ds(cc * 16, 16)
                            o_[r, s] = a_[r, s] * d_[r, s]
                else:
                    g_, c_ = gv[slot], cvb[slot]

                    @pl.loop(0, blk)
                    def _(r):
                        for cc in range(hv):
                            s = pl.ds(cc * 16, 16)
                            s2 = pl.ds(hh + cc * 16, 16)
                            s3 = pl.ds(2 * hh + cc * 16, 16)
                            o_[r, s] = (g_[r, s] * c_[r, s] * g_[r, s3]
                                        + g_[r, s2] * c_[r, s2])

                @pl.loop(0, blk // 16)
                def _(k):
                    s = pl.ds(k * 16, 16)
                    isc[slot][s] = ji[slot][1, s]

                pltpu.async_copy(o_, acc.at[isc[slot]], ssc[slot],
                                 add=True)

            issue(0, 0)

            @pl.loop(0, nblk, step=2)
            def _(b):
                issue(b + 1, 1)
                process(b, 0)

                @pl.when(b + 2 < nblk)
                def _():
                    issue(b + 2, 0)

                process(b + 1, 1)

            pltpu.make_async_copy(ov[0], acc.at[isc[0]], ssc[0]).wait()
            pltpu.make_async_copy(ov[1], acc.at[isc[1]], ssc[1]).wait()
            plsc.subcore_barrier()

            def _flush(nrows):
                if q == 0:
                    pltpu.sync_copy(acc.at[pl.ds(r0, nrows)],
                                    outs_hbm.at[core, pl.ds(r0, nrows)])
                else:
                    pltpu.sync_copy(acc.at[pl.ds(r0, nrows)],
                                    outv_hbm.at[q - 1, core,
                                                pl.ds(r0, nrows)])

            @pl.when(sub < NS - 1)
            def _():
                _flush(rows_t)

            @pl.when(sub == NS - 1)
            def _():
                _flush(last)

            plsc.subcore_barrier()

    dma = pltpu.SemaphoreType.DMA
    return pl.kernel(
        body,
        out_type=[
            jax.ShapeDtypeStruct((2, n, hh), F32),
            jax.ShapeDtypeStruct((3, 2, n, hh), F32),
        ],
        mesh=_sc_mesh(),
        compiler_params=pltpu.CompilerParams(use_tc_tiling_on_sc=False),
        scratch_types=(
            [pltpu.VMEM((2, blk), jnp.int32)] * 2
            + [pltpu.VMEM((blk,), jnp.int32)] * 2
            + [pltpu.VMEM((blk, 3 * hh), F32)] * 2
            + [pltpu.VMEM((blk, 2 * hh), F32)] * 2
            + [pltpu.VMEM((blk, hh), F32)] * 6
            + [dma] * 4
            + [pltpu.VMEM_SHARED((n_pad, hh), F32)]
        ),
    )


def _k2a_body(ns_ref, w1_ref, b1_ref, w2_ref, b2_ref, out_ref):
    x = ns_ref[...]
    y = lax.dot_general(x, w1_ref[...], (((1,), (1,)), ((), ())),
                        preferred_element_type=F32)
    hmid = _ssilu(y + b1_ref[...])
    z = lax.dot_general(hmid, w2_ref[...], (((1,), (1,)), ((), ())),
                        preferred_element_type=F32)
    z = z + b2_ref[...]
    hh = out_ref.shape[3]
    for t in range(3):
        for half in range(2):
            lo = t * 2 * hh + half * hh
            out_ref[t, half] = z[:, lo:lo + hh]


def _k2b_body(p4_ref, out_ref):
    p = p4_ref[...]
    sums = p[:, 0:3]
    counts = p[:, 3:4]
    rv = sums / jnp.maximum(counts, 1.0)
    nrm = jnp.sqrt(jnp.sum(rv * rv, axis=-1, keepdims=True) + 1e-9)
    rv = rv / nrm
    col = lax.broadcasted_iota(jnp.int32, rv.shape, 1)
    default = jnp.where(col == 0, 1.0, 0.0).astype(rv.dtype)
    rv = jnp.where(nrm < 5e-5, default, rv)
    w = out_ref.shape[1]
    out_ref[...] = jnp.concatenate(
        [rv, jnp.zeros((rv.shape[0], w - 3), rv.dtype)], axis=1)


def _make_k4_body(e_tot, h, inv_sqrt_h):
    def body(rbf_ref, rvi_ref, ud_ref, we_ref, be_ref, wgt_ref, bg_ref,
             g_ref, bt_ref, cs_ref, cv_ref):
        eb = pl.program_id(0)
        be_blk = rbf_ref.shape[0]
        ud = ud_ref[...]
        t = jnp.sum(ud * rvi_ref[:, 0:3], axis=-1, keepdims=True)
        p2 = 1.5 * t * t - 0.5
        mu = (1.0 + t + p2) / 3.0
        d0 = 1.0 - mu
        d1 = t - mu
        d2 = p2 - mu
        var = (d0 * d0 + d1 * d1 + d2 * d2) / 3.0
        inv = lax.rsqrt(var + 1e-5)
        ln0 = d0 * inv * g_ref[0, 0] + bt_ref[0, 0]
        ln1 = d1 * inv * g_ref[0, 1] + bt_ref[0, 1]
        ln2 = d2 * inv * g_ref[0, 2] + bt_ref[0, 2]
        u = (t * wgt_ref[0:1, :] + ln0 * wgt_ref[1:2, :]
             + ln1 * wgt_ref[2:3, :] + ln2 * wgt_ref[3:4, :] + bg_ref[...])
        gate = jnp.tanh(_ssilu(u))
        rbf_h = lax.dot_general(rbf_ref[...], we_ref[...],
                                (((1,), (1,)), ((), ())),
                                preferred_element_type=F32)
        rbf_h = rbf_h + be_ref[...]
        c = rbf_h * (1.0 + gate) * INV_SQRT_3
        row = eb * be_blk + lax.broadcasted_iota(jnp.int32, (be_blk, 1), 0)
        c = jnp.where(row < e_tot, c, 0.0)
        hh = h // 2
        c3 = c[:, 2 * h:3 * h]
        c1 = c[:, 0:h] * inv_sqrt_h
        c2 = c[:, h:2 * h] * inv_sqrt_h
        for half in range(2):
            lo = half * hh
            cs_ref[half] = c3[:, lo:lo + hh]
            for d in range(3):
                cv_ref[d, half, :, 0:hh] = c1[:, lo:lo + hh]
                cv_ref[d, half, :, hh:2 * hh] = (c2[:, lo:lo + hh]
                                                 * ud[:, d:d + 1])
    return body


def _k6_body(p_ref, ds_ref, dv_ref):
    ds_ref[...] = p_ref[0, 0] + p_ref[0, 1]
    dv_ref[:, 0, :] = p_ref[1, 0] + p_ref[1, 1]
    dv_ref[:, 1, :] = p_ref[2, 0] + p_ref[2, 1]
    dv_ref[:, 2, :] = p_ref[3, 0] + p_ref[3, 1]


def kernel(node_scalar, node_vector, edge_index, edge_rbf, edge_udiff,
           W1, b1, W2, b2, We, be, gamma, beta, Wg, bg):
    n, h = node_scalar.shape
    e = edge_rbf.shape[0]
    blk = 128
    e_pad = ((e + NW * blk - 1) // (NW * blk)) * (NW * blk)
    n_pad = ((n + NS * 8 - 1) // (NS * 8)) * (NS * 8)

    w = 16
    jj = jnp.pad(edge_index[0], (0, e_pad - e)).astype(jnp.int32)
    ii = jnp.pad(edge_index[1], (0, e_pad - e)).astype(jnp.int32)
    ud4 = jnp.pad(
        jnp.concatenate([edge_udiff, jnp.ones((e, 1), F32)], axis=1),
        ((0, e_pad - e), (0, w - 4)))
    z4 = jnp.zeros((n_pad, w), F32)
    zh = jnp.zeros((n_pad, h // 2), F32)
    blk5 = 80

    part4 = _make_k1(e_pad, n_pad, blk, w)(ii, ud4, z4)

    rv4 = pl.pallas_call(
        _k2b_body,
        grid=(1,),
        in_specs=[pl.BlockSpec((n_pad, w), lambda b: (0, 0))],
        out_specs=pl.BlockSpec((n_pad, w), lambda b: (0, 0)),
        out_shape=jax.ShapeDtypeStruct((n_pad, w), F32),
    )(part4)

    bn = 1000
    sp = pl.pallas_call(
        _k2a_body,
        grid=(n // bn,),
        in_specs=[
            pl.BlockSpec((bn, h), lambda b: (b, 0)),
            pl.BlockSpec(W1.shape, lambda b: (0, 0)),
            pl.BlockSpec((1, b1.shape[0]), lambda b: (0, 0)),
            pl.BlockSpec(W2.shape, lambda b: (0, 0)),
            pl.BlockSpec((1, b2.shape[0]), lambda b: (0, 0)),
        ],
        out_specs=pl.BlockSpec((3, 2, bn, h // 2), lambda b: (0, 0, b, 0)),
        out_shape=jax.ShapeDtypeStruct((3, 2, n, h // 2), F32),
    )(node_scalar, W1, b1.reshape(1, -1), W2, b2.reshape(1, -1))
    sp = sp.reshape(3, 2 * n, h // 2)

    rvi = _make_k3(e_pad, blk, w)(ii, rv4)

    be_blk = 1024
    rbf_p = jnp.pad(edge_rbf, ((0, e_pad - e), (0, 0)))
    ud_p = jnp.pad(edge_udiff, ((0, e_pad - e), (0, 0)))
    r = edge_rbf.shape[1]
    cs = pl.pallas_call(
        _make_k4_body(e, h, 1.0 / math.sqrt(h)),
        grid=(e_pad // be_blk,),
        in_specs=[
            pl.BlockSpec((be_blk, r), lambda b: (b, 0)),
            pl.BlockSpec((be_blk, w), lambda b: (b, 0)),
            pl.BlockSpec((be_blk, 3), lambda b: (b, 0)),
            pl.BlockSpec(We.shape, lambda b: (0, 0)),
            pl.BlockSpec((1, be.shape[0]), lambda b: (0, 0)),
            pl.BlockSpec((4, Wg.shape[0]), lambda b: (0, 0)),
            pl.BlockSpec((1, bg.shape[0]), lambda b: (0, 0)),
            pl.BlockSpec((1, 3), lambda b: (0, 0)),
            pl.BlockSpec((1, 3), lambda b: (0, 0)),
        ],
        out_specs=[
            pl.BlockSpec((2, be_blk, h // 2), lambda b: (0, b, 0)),
            pl.BlockSpec((3, 2, be_blk, h), lambda b: (0, 0, b, 0)),
        ],
        out_shape=[
            jax.ShapeDtypeStruct((2, e_pad, h // 2), F32),
            jax.ShapeDtypeStruct((3, 2, e_pad, h), F32),
        ],
    )(rbf_p, rvi, ud_p, We, be.reshape(1, -1), Wg.T, bg.reshape(1, -1),
      gamma.reshape(1, -1), beta.reshape(1, -1))
    cs3 = cs[0].reshape(2 * e_pad, h // 2)
    cvt = cs[1].reshape(3, 2 * e_pad, h)

    nvc = jnp.transpose(node_vector, (1, 0, 2))
    nvt = jnp.concatenate([nvc[:, :, :h // 2], nvc[:, :, h // 2:]], axis=1)
    tcat = jnp.stack([
        jnp.concatenate([sp[0], sp[1], nvt[d]], axis=1) for d in range(3)])
    ji = jnp.stack([jj.reshape(e_pad // blk5, blk5),
                    ii.reshape(e_pad // blk5, blk5)],
                   axis=1).reshape(2 * e_pad // blk5, blk5)
    outs, outv = _make_k5(e_pad, n, n_pad, h, blk5)(
        ji, tcat[0], tcat[1], tcat[2], sp[2],
        cs3, cvt[0], cvt[1], cvt[2], zh)
    ds = jnp.concatenate([outs[0], outs[1]], axis=-1)
    dv = jnp.transpose(
        jnp.concatenate([outv[:, 0], outv[:, 1]], axis=-1), (1, 0, 2))
    return ds, dv
